# baseline (speedup 1.0000x reference)
import functools
import math

import jax
import jax.numpy as jnp
from jax.experimental import pallas as pl
from jax.experimental.pallas import tpu as pltpu

_TOPK = 8
_T = 256


def _fused_kernel(hs_ref, wq_ref, keys_ref, rel_ref, wdown_ref, wup_ref,
                  values_hbm, out_ref,
                  q_scr, qsum_scr, ksel_scr, relc_scr, vd_scr, vsel_scr, sem,
                  *, n_rows, scale):
    p = pl.program_id(0)
    i = pl.program_id(1)
    nk = keys_ref.shape[0]

    @pl.when(p == 0)
    def _qproj():
        q = jax.lax.dot_general(
            hs_ref[...], wq_ref[...], (((1,), (1,)), ((), ())),
            preferred_element_type=jnp.float32)
        q_scr[pl.ds(i * _T, _T), :] = q

        @pl.when(i == 0)
        def _init():
            qsum_scr[...] = jnp.zeros_like(qsum_scr)

        qsum_scr[...] += jnp.sum(q, axis=0, keepdims=True)

    @pl.when((p == 1) & (i == 0))
    def _route():
        scores = jax.lax.dot_general(
            qsum_scr[...], keys_ref[...], (((1,), (1,)), ((), ())),
            preferred_element_type=jnp.float32) * (1.0 / n_rows)
        scores = scores + jnp.log(jnp.clip(rel_ref[...], 1e-10))
        iota = jax.lax.broadcasted_iota(jnp.int32, (1, nk), 1)
        masks = []
        for _ in range(_TOPK):
            m = jnp.max(scores, axis=1, keepdims=True)
            idxv = jnp.min(jnp.where(scores == m, iota, nk), axis=1,
                           keepdims=True)
            mj = iota == idxv
            masks.append(mj)
            scores = jnp.where(mj, -jnp.inf, scores)
        onehot = jnp.concatenate(
            [mj.astype(jnp.float32) for mj in masks], axis=0)
        ksel_scr[...] = jax.lax.dot_general(
            onehot, keys_ref[...], (((1,), (0,)), ((), ())),
            preferred_element_type=jnp.float32)
        relc_scr[...] = jnp.clip(
            jax.lax.dot_general(
                onehot, rel_ref[...], (((1,), (1,)), ((), ())),
                preferred_element_type=jnp.float32), 1e-10, None)
        copies = []
        for j, mj in enumerate(masks):
            idx = jnp.min(jnp.where(mj, iota, nk))
            cp = pltpu.make_async_copy(
                values_hbm.at[pl.ds(idx, 1), :], vsel_scr.at[pl.ds(j, 1), :],
                sem)
            cp.start()
            copies.append(cp)
        for cp in copies:
            cp.wait()
        vd = jax.lax.dot_general(
            vsel_scr[...], wdown_ref[...], (((1,), (1,)), ((), ())),
            preferred_element_type=jnp.float32)
        vd_scr[...] = vd * relc_scr[...]

    @pl.when(p == 1)
    def _attn():
        q = q_scr[pl.ds(i * _T, _T), :]
        s = jax.lax.dot_general(
            q, ksel_scr[...], (((1,), (1,)), ((), ())),
            preferred_element_type=jnp.float32) * scale
        m = jnp.max(s, axis=-1, keepdims=True)
        e = jnp.exp(s - m)
        denom = jax.lax.dot_general(
            e, relc_scr[...], (((1,), (0,)), ((), ())),
            preferred_element_type=jnp.float32)
        u = jax.lax.dot_general(
            e, vd_scr[...], (((1,), (0,)), ((), ())),
            preferred_element_type=jnp.float32)
        mid = u / denom
        g = mid * 0.5 * (1.0 + jax.lax.erf(mid * (1.0 / math.sqrt(2.0))))
        out_ref[...] = jax.lax.dot_general(
            g, wup_ref[...], (((1,), (1,)), ((), ())),
            preferred_element_type=jnp.float32)


def kernel(hidden_states, keys, values, reliability, Wq, Wdown, Wup):
    b, s, h = hidden_states.shape
    n = b * s
    nk, dk = keys.shape
    dv = Wdown.shape[0]
    hs = hidden_states.reshape(n, h)
    rel_row = reliability.reshape(1, nk)
    nt = n // _T

    out = pl.pallas_call(
        functools.partial(_fused_kernel, n_rows=n, scale=1.0 / math.sqrt(dk)),
        grid=(2, nt),
        in_specs=[
            pl.BlockSpec((_T, h), lambda p, i: (jnp.where(p == 0, i, nt - 1), 0)),
            pl.BlockSpec((dk, h), lambda p, i: (0, 0)),
            pl.BlockSpec((nk, dk), lambda p, i: (0, 0)),
            pl.BlockSpec((1, nk), lambda p, i: (0, 0)),
            pl.BlockSpec((dv, h), lambda p, i: (0, 0)),
            pl.BlockSpec((h, dv), lambda p, i: (0, 0)),
            pl.BlockSpec(memory_space=pl.ANY),
        ],
        out_specs=pl.BlockSpec((_T, h), lambda p, i: (jnp.where(p == 0, 0, i), 0)),
        out_shape=jax.ShapeDtypeStruct((n, h), jnp.float32),
        scratch_shapes=[
            pltpu.VMEM((n, dk), jnp.float32),
            pltpu.VMEM((1, dk), jnp.float32),
            pltpu.VMEM((_TOPK, dk), jnp.float32),
            pltpu.VMEM((_TOPK, 1), jnp.float32),
            pltpu.VMEM((_TOPK, dv), jnp.float32),
            pltpu.VMEM((_TOPK, h), jnp.float32),
            pltpu.SemaphoreType.DMA,
        ],
        compiler_params=pltpu.CompilerParams(
            vmem_limit_bytes=62 * 1024 * 1024),
    )(hs, Wq, keys, rel_row, Wdown, Wup, values)

    return out.reshape(b, s, h)


# speedup vs baseline: 1.1672x; 1.1672x over previous
import functools
import math

import jax
import jax.numpy as jnp
from jax.experimental import pallas as pl
from jax.experimental.pallas import tpu as pltpu

_TOPK = 8
_T0 = 512
_T1 = 1024


def _fused_kernel(hs_ref, wq_ref, keys_ref, rel_ref, wdown_ref, wup_ref,
                  values_hbm, out_ref,
                  q_scr, qsum_scr, ksel_scr, relc_scr, vd_scr, vsel_scr, sem,
                  *, n_rows, scale, nt0):
    s_id = pl.program_id(0)
    nk = keys_ref.shape[0]

    @pl.when(s_id < nt0)
    def _qproj():
        q = jax.lax.dot_general(
            hs_ref[...], wq_ref[...], (((1,), (1,)), ((), ())),
            preferred_element_type=jnp.float32)
        q_scr[pl.ds(s_id * _T0, _T0), :] = q

        @pl.when(s_id == 0)
        def _init():
            qsum_scr[...] = jnp.zeros_like(qsum_scr)

        qsum_scr[...] += jnp.sum(q, axis=0, keepdims=True)

    @pl.when(s_id == nt0)
    def _route():
        scores = jax.lax.dot_general(
            qsum_scr[...], keys_ref[...], (((1,), (1,)), ((), ())),
            preferred_element_type=jnp.float32) * (1.0 / n_rows)
        scores = scores + jnp.log(jnp.clip(rel_ref[...], 1e-10))
        iota = jax.lax.broadcasted_iota(jnp.int32, (1, nk), 1)
        masks = []
        for _ in range(_TOPK):
            m = jnp.max(scores, axis=1, keepdims=True)
            idxv = jnp.min(jnp.where(scores == m, iota, nk), axis=1,
                           keepdims=True)
            mj = iota == idxv
            masks.append(mj)
            scores = jnp.where(mj, -jnp.inf, scores)
        onehot = jnp.concatenate(
            [mj.astype(jnp.float32) for mj in masks], axis=0)
        ksel_scr[...] = jax.lax.dot_general(
            onehot, keys_ref[...], (((1,), (0,)), ((), ())),
            preferred_element_type=jnp.float32)
        relc_scr[...] = jnp.clip(
            jax.lax.dot_general(
                onehot, rel_ref[...], (((1,), (1,)), ((), ())),
                preferred_element_type=jnp.float32), 1e-10, None)
        copies = []
        for j, mj in enumerate(masks):
            idx = jnp.min(jnp.where(mj, iota, nk))
            cp = pltpu.make_async_copy(
                values_hbm.at[pl.ds(idx, 1), :], vsel_scr.at[pl.ds(j, 1), :],
                sem)
            cp.start()
            copies.append(cp)
        for cp in copies:
            cp.wait()
        vd = jax.lax.dot_general(
            vsel_scr[...], wdown_ref[...], (((1,), (1,)), ((), ())),
            preferred_element_type=jnp.float32)
        vd_scr[...] = vd * relc_scr[...]

    @pl.when(s_id >= nt0)
    def _attn():
        q = q_scr[pl.ds((s_id - nt0) * _T1, _T1), :]
        s = jax.lax.dot_general(
            q, ksel_scr[...], (((1,), (1,)), ((), ())),
            preferred_element_type=jnp.float32) * scale
        m = jnp.max(s, axis=-1, keepdims=True)
        e = jnp.exp(s - m)
        denom = jax.lax.dot_general(
            e, relc_scr[...], (((1,), (0,)), ((), ())),
            preferred_element_type=jnp.float32)
        u = jax.lax.dot_general(
            e, vd_scr[...], (((1,), (0,)), ((), ())),
            preferred_element_type=jnp.float32)
        mid = u / denom
        g = mid * 0.5 * (1.0 + jax.lax.erf(mid * (1.0 / math.sqrt(2.0))))
        out_ref[...] = jax.lax.dot_general(
            g, wup_ref[...], (((1,), (1,)), ((), ())),
            preferred_element_type=jnp.float32)


def kernel(hidden_states, keys, values, reliability, Wq, Wdown, Wup):
    b, s, h = hidden_states.shape
    n = b * s
    nk, dk = keys.shape
    dv = Wdown.shape[0]
    hs = hidden_states.reshape(n, h)
    rel_row = reliability.reshape(1, nk)
    nt0 = n // _T0
    nt1 = n // _T1

    out = pl.pallas_call(
        functools.partial(_fused_kernel, n_rows=n, scale=1.0 / math.sqrt(dk),
                          nt0=nt0),
        grid=(nt0 + nt1,),
        in_specs=[
            pl.BlockSpec((_T0, h), lambda s: (jnp.minimum(s, nt0 - 1), 0)),
            pl.BlockSpec((dk, h), lambda s: (0, 0)),
            pl.BlockSpec((nk, dk), lambda s: (0, 0)),
            pl.BlockSpec((1, nk), lambda s: (0, 0)),
            pl.BlockSpec((dv, h), lambda s: (0, 0)),
            pl.BlockSpec((h, dv), lambda s: (0, 0)),
            pl.BlockSpec(memory_space=pl.ANY),
        ],
        out_specs=pl.BlockSpec(
            (_T1, h), lambda s: (jnp.maximum(s - nt0, 0), 0)),
        out_shape=jax.ShapeDtypeStruct((n, h), jnp.float32),
        scratch_shapes=[
            pltpu.VMEM((n, dk), jnp.float32),
            pltpu.VMEM((1, dk), jnp.float32),
            pltpu.VMEM((_TOPK, dk), jnp.float32),
            pltpu.VMEM((_TOPK, 1), jnp.float32),
            pltpu.VMEM((_TOPK, dv), jnp.float32),
            pltpu.VMEM((_TOPK, h), jnp.float32),
            pltpu.SemaphoreType.DMA,
        ],
        compiler_params=pltpu.CompilerParams(
            vmem_limit_bytes=63 * 1024 * 1024),
    )(hs, Wq, keys, rel_row, Wdown, Wup, values)

    return out.reshape(b, s, h)


# speedup vs baseline: 1.1853x; 1.0155x over previous
import functools
import math

import jax
import jax.numpy as jnp
from jax.experimental import pallas as pl
from jax.experimental.pallas import tpu as pltpu

_TOPK = 8
_T = 512


def _fused_kernel(hs_ref, wq_ref, keys_ref, rel_ref, wdown_ref, wup_ref,
                  values_hbm, out_ref,
                  q_scr, qsum_scr, ksel_scr, relc_scr, vd_scr, vsel_scr, sem,
                  *, n_rows, scale):
    p = pl.program_id(0)
    i = pl.program_id(1)
    nk = keys_ref.shape[0]

    @pl.when(p == 0)
    def _qproj():
        q = jax.lax.dot_general(
            hs_ref[...], wq_ref[...], (((1,), (1,)), ((), ())),
            preferred_element_type=jnp.float32)
        q_scr[pl.ds(i * _T, _T), :] = q

        @pl.when(i == 0)
        def _init():
            qsum_scr[...] = jnp.zeros_like(qsum_scr)

        qsum_scr[...] += jnp.sum(q, axis=0, keepdims=True)

    @pl.when((p == 1) & (i == 0))
    def _route():
        scores = jax.lax.dot_general(
            qsum_scr[...], keys_ref[...], (((1,), (1,)), ((), ())),
            preferred_element_type=jnp.float32) * (1.0 / n_rows)
        scores = scores + jnp.log(jnp.clip(rel_ref[...], 1e-10))
        iota = jax.lax.broadcasted_iota(jnp.int32, (1, nk), 1)
        masks = []
        for _ in range(_TOPK):
            m = jnp.max(scores, axis=1, keepdims=True)
            idxv = jnp.min(jnp.where(scores == m, iota, nk), axis=1,
                           keepdims=True)
            mj = iota == idxv
            masks.append(mj)
            scores = jnp.where(mj, -jnp.inf, scores)
        onehot = jnp.concatenate(
            [mj.astype(jnp.float32) for mj in masks], axis=0)
        ksel_scr[...] = jax.lax.dot_general(
            onehot, keys_ref[...], (((1,), (0,)), ((), ())),
            preferred_element_type=jnp.float32)
        relc_scr[...] = jnp.clip(
            jax.lax.dot_general(
                onehot, rel_ref[...], (((1,), (1,)), ((), ())),
                preferred_element_type=jnp.float32), 1e-10, None)
        copies = []
        for j, mj in enumerate(masks):
            idx = jnp.min(jnp.where(mj, iota, nk))
            cp = pltpu.make_async_copy(
                values_hbm.at[pl.ds(idx, 1), :], vsel_scr.at[pl.ds(j, 1), :],
                sem)
            cp.start()
            copies.append(cp)
        for cp in copies:
            cp.wait()
        vd = jax.lax.dot_general(
            vsel_scr[...], wdown_ref[...], (((1,), (1,)), ((), ())),
            preferred_element_type=jnp.float32)
        vd_scr[...] = vd * relc_scr[...]

    @pl.when(p == 1)
    def _attn():
        q = q_scr[pl.ds(i * _T, _T), :]
        s = jax.lax.dot_general(
            q, ksel_scr[...], (((1,), (1,)), ((), ())),
            preferred_element_type=jnp.float32) * scale
        m = jnp.max(s, axis=-1, keepdims=True)
        e = jnp.exp(s - m)
        denom = jax.lax.dot_general(
            e, relc_scr[...], (((1,), (0,)), ((), ())),
            preferred_element_type=jnp.float32)
        u = jax.lax.dot_general(
            e, vd_scr[...], (((1,), (0,)), ((), ())),
            preferred_element_type=jnp.float32)
        mid = u / denom
        g = mid * 0.5 * (1.0 + jax.lax.erf(mid * (1.0 / math.sqrt(2.0))))
        out_ref[...] = jax.lax.dot_general(
            g, wup_ref[...], (((1,), (1,)), ((), ())),
            preferred_element_type=jnp.float32)


def kernel(hidden_states, keys, values, reliability, Wq, Wdown, Wup):
    b, s, h = hidden_states.shape
    n = b * s
    nk, dk = keys.shape
    dv = Wdown.shape[0]
    hs = hidden_states.reshape(n, h)
    rel_row = reliability.reshape(1, nk)
    nt = n // _T

    out = pl.pallas_call(
        functools.partial(_fused_kernel, n_rows=n, scale=1.0 / math.sqrt(dk)),
        grid=(2, nt),
        in_specs=[
            pl.BlockSpec((_T, h), lambda p, i: (jnp.where(p == 0, i, nt - 1), 0)),
            pl.BlockSpec((dk, h), lambda p, i: (0, 0)),
            pl.BlockSpec((nk, dk), lambda p, i: (0, 0)),
            pl.BlockSpec((1, nk), lambda p, i: (0, 0)),
            pl.BlockSpec((dv, h), lambda p, i: (0, 0)),
            pl.BlockSpec((h, dv), lambda p, i: (0, 0)),
            pl.BlockSpec(memory_space=pl.ANY),
        ],
        out_specs=pl.BlockSpec((_T, h), lambda p, i: (jnp.where(p == 0, 0, i), 0)),
        out_shape=jax.ShapeDtypeStruct((n, h), jnp.float32),
        scratch_shapes=[
            pltpu.VMEM((n, dk), jnp.float32),
            pltpu.VMEM((1, dk), jnp.float32),
            pltpu.VMEM((_TOPK, dk), jnp.float32),
            pltpu.VMEM((_TOPK, 1), jnp.float32),
            pltpu.VMEM((_TOPK, dv), jnp.float32),
            pltpu.VMEM((_TOPK, h), jnp.float32),
            pltpu.SemaphoreType.DMA,
        ],
        compiler_params=pltpu.CompilerParams(
            vmem_limit_bytes=63 * 1024 * 1024),
    )(hs, Wq, keys, rel_row, Wdown, Wup, values)

    return out.reshape(b, s, h)
